# baseline (speedup 1.0000x reference)
import functools

import jax
import jax.numpy as jnp
from jax import lax
from jax.experimental import pallas as pl
from jax.experimental.pallas import tpu as pltpu
from jax.experimental.pallas import tpu_sc as plsc

N_POI = 1000
N_POI_PAD = 1024
N_USERS = 100
DIM = 64
BATCH = 16384
NUM_TILES = 32
NUM_DG = 4
NUM_EG = NUM_TILES // NUM_DG
DPG = DIM // NUM_DG
GSTRIDE = DPG + 1
B_PER_TILE = BATCH // NUM_EG
CHUNKS = B_PER_TILE // 16
SLAB = N_POI_PAD * GSTRIDE
N_USERS_PAD = 104
VSLAB = N_USERS_PAD * GSTRIDE


def _sc_body(poi_h, vtab_h, bias_h, uid_h, pid_h, qid_h, nid_h,
             d2p_h, d2n_h, bd_h, mask_h,
             poi_v, vtab_v, bias_v, uid_v, pid_v, qid_v, nid_v,
             outp_v, outn_v, outb_v, mask_v, sem):
  c = lax.axis_index("c")
  s = lax.axis_index("s")
  wid = s * 2 + c
  g = lax.rem(wid, NUM_DG)
  e = wid // NUM_DG
  ebase = e * B_PER_TILE
  obase = g * BATCH + ebase

  with jax.named_scope("stage"):
    cps = [
        pltpu.async_copy(poi_h.at[pl.ds(g * SLAB, SLAB)], poi_v, sem),
        pltpu.async_copy(vtab_h.at[pl.ds(g * VSLAB, VSLAB)], vtab_v, sem),
        pltpu.async_copy(bias_h, bias_v, sem),
        pltpu.async_copy(uid_h.at[pl.ds(ebase, B_PER_TILE)], uid_v, sem),
        pltpu.async_copy(pid_h.at[pl.ds(ebase, B_PER_TILE)], pid_v, sem),
        pltpu.async_copy(qid_h.at[pl.ds(ebase, B_PER_TILE)], qid_v, sem),
        pltpu.async_copy(nid_h.at[pl.ds(ebase, B_PER_TILE)], nid_v, sem),
    ]
    zeros16 = jnp.zeros((16,), jnp.float32)
    for i in range(N_POI_PAD // 16):
      mask_v[pl.ds(i * 16, 16)] = zeros16
    for cp in cps:
      cp.wait()

  ones16 = jnp.ones((16,), jnp.float32)
  is_g0 = g == 0

  scope = jax.named_scope("gatherloop")
  scope.__enter__()

  @plsc.parallel_loop(0, CHUNKS, unroll=4)
  def chunk(i):
    sl = pl.ds(i * 16, 16)
    u = uid_v[sl]
    p = pid_v[sl]
    q = qid_v[sl]
    r = nid_v[sl]
    ub = u * GSTRIDE
    pb = p * GSTRIDE
    qb = q * GSTRIDE
    rb = r * GSTRIDE
    accp0 = jnp.zeros((16,), jnp.float32)
    accp1 = jnp.zeros((16,), jnp.float32)
    accn0 = jnp.zeros((16,), jnp.float32)
    accn1 = jnp.zeros((16,), jnp.float32)
    for d in range(0, DPG, 2):
      td0 = plsc.load_gather(poi_v, [pb + d]) + plsc.load_gather(vtab_v, [ub + d])
      ep0 = td0 - plsc.load_gather(poi_v, [qb + d])
      en0 = td0 - plsc.load_gather(poi_v, [rb + d])
      accp0 = accp0 + ep0 * ep0
      accn0 = accn0 + en0 * en0
      td1 = plsc.load_gather(poi_v, [pb + (d + 1)]) + plsc.load_gather(
          vtab_v, [ub + (d + 1)])
      ep1 = td1 - plsc.load_gather(poi_v, [qb + (d + 1)])
      en1 = td1 - plsc.load_gather(poi_v, [rb + (d + 1)])
      accp1 = accp1 + ep1 * ep1
      accn1 = accn1 + en1 * en1
    outp_v[sl] = accp0 + accp1
    outn_v[sl] = accn0 + accn1

  @pl.when(is_g0)
  def _():
    @plsc.parallel_loop(0, CHUNKS, unroll=4)
    def chunk2(i):
      sl = pl.ds(i * 16, 16)
      p = pid_v[sl]
      q = qid_v[sl]
      r = nid_v[sl]
      bq = plsc.load_gather(bias_v, [q])
      br = plsc.load_gather(bias_v, [r])
      outb_v[sl] = bq - br
      plsc.store_scatter(mask_v, [p], ones16)
      plsc.store_scatter(mask_v, [q], ones16)
      plsc.store_scatter(mask_v, [r], ones16)

  scope.__exit__(None, None, None)
  pltpu.sync_copy(outp_v, d2p_h.at[pl.ds(obase, B_PER_TILE)])
  pltpu.sync_copy(outn_v, d2n_h.at[pl.ds(obase, B_PER_TILE)])

  @pl.when(is_g0)
  def _():
    pltpu.sync_copy(outb_v, bd_h.at[pl.ds(ebase, B_PER_TILE)])
    pltpu.sync_copy(mask_v, mask_h.at[e])


_sc_kernel = functools.partial(
    pl.kernel,
    out_type=(
        jax.ShapeDtypeStruct((NUM_DG * BATCH,), jnp.float32),
        jax.ShapeDtypeStruct((NUM_DG * BATCH,), jnp.float32),
        jax.ShapeDtypeStruct((BATCH,), jnp.float32),
        jax.ShapeDtypeStruct((NUM_EG, N_POI_PAD), jnp.float32),
    ),
    mesh=plsc.VectorSubcoreMesh(core_axis_name="c", subcore_axis_name="s"),
    compiler_params=pltpu.CompilerParams(needs_layout_passes=False),
    scratch_types=[
        pltpu.VMEM((SLAB,), jnp.float32),
        pltpu.VMEM((VSLAB,), jnp.float32),
        pltpu.VMEM((N_POI_PAD,), jnp.float32),
        pltpu.VMEM((B_PER_TILE,), jnp.int32),
        pltpu.VMEM((B_PER_TILE,), jnp.int32),
        pltpu.VMEM((B_PER_TILE,), jnp.int32),
        pltpu.VMEM((B_PER_TILE,), jnp.int32),
        pltpu.VMEM((B_PER_TILE,), jnp.float32),
        pltpu.VMEM((B_PER_TILE,), jnp.float32),
        pltpu.VMEM((B_PER_TILE,), jnp.float32),
        pltpu.VMEM((N_POI_PAD,), jnp.float32),
        pltpu.SemaphoreType.DMA,
    ],
)(_sc_body)


def _tc_body(poi_ref, masks_ref, d2p_ref, d2n_ref, bd_ref, w_ref, obj_ref):
  m = jnp.max(masks_ref[...], axis=0)[:N_POI]
  poi = poi_ref[...]
  n2 = jnp.sum(poi * poi, axis=1)
  denom = jnp.maximum(1.0, jnp.sqrt(n2))
  scale = jnp.where(m > 0.0, 1.0 / denom, 1.0)
  w_ref[...] = poi * scale[:, None]
  d2p = (d2p_ref[pl.ds(0, BATCH)] + d2p_ref[pl.ds(BATCH, BATCH)] +
         d2p_ref[pl.ds(2 * BATCH, BATCH)] + d2p_ref[pl.ds(3 * BATCH, BATCH)])
  d2n = (d2n_ref[pl.ds(0, BATCH)] + d2n_ref[pl.ds(BATCH, BATCH)] +
         d2n_ref[pl.ds(2 * BATCH, BATCH)] + d2n_ref[pl.ds(3 * BATCH, BATCH)])
  obj_ref[...] = bd_ref[...] - jnp.sqrt(d2p) + jnp.sqrt(d2n)


def _slab_layout(t, rows_pad):
  t = jnp.pad(t, ((0, rows_pad - t.shape[0]), (0, 0)))
  t = t.reshape(rows_pad, NUM_DG, DPG)
  t = jnp.pad(t, ((0, 0), (0, 0), (0, GSTRIDE - DPG)))
  return t.transpose(1, 0, 2).reshape(-1)


def kernel(user_id, prev_id, pos_id, neg_id, poi_weight, user_weight,
           user_global_weight, poi_bias_weight):
  uid = user_id.astype(jnp.int32)
  pid = prev_id.astype(jnp.int32)
  qid = pos_id.astype(jnp.int32)
  nid = neg_id.astype(jnp.int32)
  poi_s = _slab_layout(poi_weight, N_POI_PAD)
  vtab_s = _slab_layout(user_weight + user_global_weight, N_USERS_PAD)
  bias_p = jnp.pad(poi_bias_weight[:, 0], (0, N_POI_PAD - N_POI))

  d2p, d2n, bd, masks = _sc_kernel(poi_s, vtab_s, bias_p, uid, pid, qid, nid)

  w, obj = pl.pallas_call(
      _tc_body,
      out_shape=(
          jax.ShapeDtypeStruct((N_POI, DIM), jnp.float32),
          jax.ShapeDtypeStruct((BATCH,), jnp.float32),
      ),
  )(poi_weight, masks, d2p, d2n, bd)

  return obj, w


# speedup vs baseline: 1.2073x; 1.2073x over previous
import functools

import jax
import jax.numpy as jnp
from jax import lax
from jax.experimental import pallas as pl
from jax.experimental.pallas import tpu as pltpu
from jax.experimental.pallas import tpu_sc as plsc

N_POI = 1000
N_POI_PAD = 1024
N_USERS = 100
DIM = 64
BATCH = 16384
NUM_TILES = 32
NUM_DG = 4
NUM_EG = NUM_TILES // NUM_DG
DPG = DIM // NUM_DG
GSTRIDE = DPG + 1
B_PER_TILE = BATCH // NUM_EG
CHUNKS = B_PER_TILE // 16
SLAB = N_POI_PAD * GSTRIDE
N_USERS_PAD = 104
VSLAB = N_USERS_PAD * GSTRIDE


def _sc_body(poi_h, vtab_h, bias_h, uid_h, pid_h, qid_h, nid_h,
             d2p_h, d2n_h, bd_h, mask_h,
             poi_v, vtab_v, bias_v, uid_v, pid_v, qid_v, nid_v,
             outp_v, outn_v, outb_v, mask_v, sem):
  c = lax.axis_index("c")
  s = lax.axis_index("s")
  wid = s * 2 + c
  g = lax.rem(wid, NUM_DG)
  e = wid // NUM_DG
  ebase = e * B_PER_TILE
  obase = g * BATCH + ebase

  with jax.named_scope("stage"):
    cps = [
        pltpu.async_copy(poi_h.at[pl.ds(g * SLAB, SLAB)], poi_v, sem),
        pltpu.async_copy(vtab_h.at[pl.ds(g * VSLAB, VSLAB)], vtab_v, sem),
        pltpu.async_copy(bias_h, bias_v, sem),
        pltpu.async_copy(uid_h.at[pl.ds(ebase, B_PER_TILE)], uid_v, sem),
        pltpu.async_copy(pid_h.at[pl.ds(ebase, B_PER_TILE)], pid_v, sem),
        pltpu.async_copy(qid_h.at[pl.ds(ebase, B_PER_TILE)], qid_v, sem),
        pltpu.async_copy(nid_h.at[pl.ds(ebase, B_PER_TILE)], nid_v, sem),
    ]
    zeros16 = jnp.zeros((16,), jnp.float32)
    for i in range(N_POI_PAD // 16):
      mask_v[pl.ds(i * 16, 16)] = zeros16
    for cp in cps:
      cp.wait()

  ones16 = jnp.ones((16,), jnp.float32)
  is_g0 = g == 0

  scope = jax.named_scope("gatherloop")
  scope.__enter__()

  @plsc.parallel_loop(0, CHUNKS, unroll=2)
  def chunk(i):
    sl = pl.ds(i * 16, 16)
    u = uid_v[sl]
    p = pid_v[sl]
    q = qid_v[sl]
    r = nid_v[sl]
    ub = u * GSTRIDE
    pb = p * GSTRIDE
    qb = q * GSTRIDE
    rb = r * GSTRIDE
    accp0 = jnp.zeros((16,), jnp.float32)
    accp1 = jnp.zeros((16,), jnp.float32)
    accn0 = jnp.zeros((16,), jnp.float32)
    accn1 = jnp.zeros((16,), jnp.float32)
    for d in range(0, DPG, 2):
      td0 = plsc.load_gather(poi_v, [pb + d]) + plsc.load_gather(vtab_v, [ub + d])
      ep0 = td0 - plsc.load_gather(poi_v, [qb + d])
      en0 = td0 - plsc.load_gather(poi_v, [rb + d])
      accp0 = accp0 + ep0 * ep0
      accn0 = accn0 + en0 * en0
      td1 = plsc.load_gather(poi_v, [pb + (d + 1)]) + plsc.load_gather(
          vtab_v, [ub + (d + 1)])
      ep1 = td1 - plsc.load_gather(poi_v, [qb + (d + 1)])
      en1 = td1 - plsc.load_gather(poi_v, [rb + (d + 1)])
      accp1 = accp1 + ep1 * ep1
      accn1 = accn1 + en1 * en1
    outp_v[sl] = accp0 + accp1
    outn_v[sl] = accn0 + accn1

  @pl.when(is_g0)
  def _():
    @plsc.parallel_loop(0, CHUNKS, unroll=4)
    def chunk2(i):
      sl = pl.ds(i * 16, 16)
      p = pid_v[sl]
      q = qid_v[sl]
      r = nid_v[sl]
      bq = plsc.load_gather(bias_v, [q])
      br = plsc.load_gather(bias_v, [r])
      outb_v[sl] = bq - br
      plsc.store_scatter(mask_v, [p], ones16)
      plsc.store_scatter(mask_v, [q], ones16)
      plsc.store_scatter(mask_v, [r], ones16)

  scope.__exit__(None, None, None)
  pltpu.sync_copy(outp_v, d2p_h.at[pl.ds(obase, B_PER_TILE)])
  pltpu.sync_copy(outn_v, d2n_h.at[pl.ds(obase, B_PER_TILE)])

  @pl.when(is_g0)
  def _():
    pltpu.sync_copy(outb_v, bd_h.at[pl.ds(ebase, B_PER_TILE)])
    pltpu.sync_copy(mask_v, mask_h.at[e])


_sc_kernel = functools.partial(
    pl.kernel,
    out_type=(
        jax.ShapeDtypeStruct((NUM_DG * BATCH,), jnp.float32),
        jax.ShapeDtypeStruct((NUM_DG * BATCH,), jnp.float32),
        jax.ShapeDtypeStruct((BATCH,), jnp.float32),
        jax.ShapeDtypeStruct((NUM_EG, N_POI_PAD), jnp.float32),
    ),
    mesh=plsc.VectorSubcoreMesh(core_axis_name="c", subcore_axis_name="s"),
    compiler_params=pltpu.CompilerParams(needs_layout_passes=False),
    scratch_types=[
        pltpu.VMEM((SLAB,), jnp.float32),
        pltpu.VMEM((VSLAB,), jnp.float32),
        pltpu.VMEM((N_POI_PAD,), jnp.float32),
        pltpu.VMEM((B_PER_TILE,), jnp.int32),
        pltpu.VMEM((B_PER_TILE,), jnp.int32),
        pltpu.VMEM((B_PER_TILE,), jnp.int32),
        pltpu.VMEM((B_PER_TILE,), jnp.int32),
        pltpu.VMEM((B_PER_TILE,), jnp.float32),
        pltpu.VMEM((B_PER_TILE,), jnp.float32),
        pltpu.VMEM((B_PER_TILE,), jnp.float32),
        pltpu.VMEM((N_POI_PAD,), jnp.float32),
        pltpu.SemaphoreType.DMA,
    ],
)(_sc_body)


def _tc_body(poi_ref, masks_ref, d2p_ref, d2n_ref, bd_ref, w_ref, obj_ref):
  m = jnp.max(masks_ref[...], axis=0)[:N_POI]
  poi = poi_ref[...]
  n2 = jnp.sum(poi * poi, axis=1)
  denom = jnp.maximum(1.0, jnp.sqrt(n2))
  scale = jnp.where(m > 0.0, 1.0 / denom, 1.0)
  w_ref[...] = poi * scale[:, None]
  d2p = (d2p_ref[pl.ds(0, BATCH)] + d2p_ref[pl.ds(BATCH, BATCH)] +
         d2p_ref[pl.ds(2 * BATCH, BATCH)] + d2p_ref[pl.ds(3 * BATCH, BATCH)])
  d2n = (d2n_ref[pl.ds(0, BATCH)] + d2n_ref[pl.ds(BATCH, BATCH)] +
         d2n_ref[pl.ds(2 * BATCH, BATCH)] + d2n_ref[pl.ds(3 * BATCH, BATCH)])
  obj_ref[...] = bd_ref[...] - jnp.sqrt(d2p) + jnp.sqrt(d2n)


def _slab_layout(t, rows_pad):
  t = jnp.pad(t, ((0, rows_pad - t.shape[0]), (0, 0)))
  t = t.reshape(rows_pad, NUM_DG, DPG)
  t = jnp.pad(t, ((0, 0), (0, 0), (0, GSTRIDE - DPG)))
  return t.transpose(1, 0, 2).reshape(-1)


def kernel(user_id, prev_id, pos_id, neg_id, poi_weight, user_weight,
           user_global_weight, poi_bias_weight):
  uid = user_id.astype(jnp.int32)
  pid = prev_id.astype(jnp.int32)
  qid = pos_id.astype(jnp.int32)
  nid = neg_id.astype(jnp.int32)
  poi_s = _slab_layout(poi_weight, N_POI_PAD)
  vtab_s = _slab_layout(user_weight + user_global_weight, N_USERS_PAD)
  bias_p = jnp.pad(poi_bias_weight[:, 0], (0, N_POI_PAD - N_POI))

  d2p, d2n, bd, masks = _sc_kernel(poi_s, vtab_s, bias_p, uid, pid, qid, nid)

  w, obj = pl.pallas_call(
      _tc_body,
      out_shape=(
          jax.ShapeDtypeStruct((N_POI, DIM), jnp.float32),
          jax.ShapeDtypeStruct((BATCH,), jnp.float32),
      ),
  )(poi_weight, masks, d2p, d2n, bd)

  return obj, w


# speedup vs baseline: 1.2628x; 1.0460x over previous
import functools

import jax
import jax.numpy as jnp
from jax import lax
from jax.experimental import pallas as pl
from jax.experimental.pallas import tpu as pltpu
from jax.experimental.pallas import tpu_sc as plsc

N_POI = 1000
N_POI_PAD = 1024
N_USERS = 100
N_USERS_PAD = 104
DIM = 64
STRIDE = 65
BATCH = 16384
NUM_TILES = 32
B_PER_TILE = BATCH // NUM_TILES
CHUNKS = B_PER_TILE // 16


def _sc_body(poi_h, vtab_h, bias_h, uid_h, pid_h, qid_h, nid_h,
             d2p_h, d2n_h, bd_h, mask_h,
             poi_v, vtab_v, bias_v, uid_v, pid_v, qid_v, nid_v,
             outp_v, outn_v, outb_v, mask_v, poi_sh, vtab_sh, bias_sh, sem):
  c = lax.axis_index("c")
  s = lax.axis_index("s")
  wid = s * 2 + c
  base = wid * B_PER_TILE

  with jax.named_scope("stage"):
    cps = [
        pltpu.async_copy(uid_h.at[pl.ds(base, B_PER_TILE)], uid_v, sem),
        pltpu.async_copy(pid_h.at[pl.ds(base, B_PER_TILE)], pid_v, sem),
        pltpu.async_copy(qid_h.at[pl.ds(base, B_PER_TILE)], qid_v, sem),
        pltpu.async_copy(nid_h.at[pl.ds(base, B_PER_TILE)], nid_v, sem),
    ]

    @pl.when(s == 0)
    def _():
      pltpu.sync_copy(poi_h, poi_sh)
      pltpu.sync_copy(vtab_h, vtab_sh)
      pltpu.sync_copy(bias_h, bias_sh)

    zeros16 = jnp.zeros((16,), jnp.float32)
    for i in range(N_POI_PAD // 16):
      mask_v[pl.ds(i * 16, 16)] = zeros16
    plsc.subcore_barrier()
    cps += [
        pltpu.async_copy(poi_sh, poi_v, sem),
        pltpu.async_copy(vtab_sh, vtab_v, sem),
        pltpu.async_copy(bias_sh, bias_v, sem),
    ]
    for cp in cps:
      cp.wait()

  ones16 = jnp.ones((16,), jnp.float32)

  scope = jax.named_scope("gatherloop")
  scope.__enter__()

  @plsc.parallel_loop(0, CHUNKS, unroll=2)
  def chunk(i):
    sl = pl.ds(i * 16, 16)
    u = uid_v[sl]
    p = pid_v[sl]
    q = qid_v[sl]
    r = nid_v[sl]
    bq = plsc.load_gather(bias_v, [q])
    br = plsc.load_gather(bias_v, [r])
    ub = u * STRIDE
    pb = p * STRIDE
    qb = q * STRIDE
    rb = r * STRIDE
    accp0 = jnp.zeros((16,), jnp.float32)
    accp1 = jnp.zeros((16,), jnp.float32)
    accn0 = jnp.zeros((16,), jnp.float32)
    accn1 = jnp.zeros((16,), jnp.float32)
    for d in range(0, DIM, 2):
      td0 = plsc.load_gather(poi_v, [pb + d]) + plsc.load_gather(vtab_v, [ub + d])
      ep0 = td0 - plsc.load_gather(poi_v, [qb + d])
      en0 = td0 - plsc.load_gather(poi_v, [rb + d])
      accp0 = accp0 + ep0 * ep0
      accn0 = accn0 + en0 * en0
      td1 = plsc.load_gather(poi_v, [pb + (d + 1)]) + plsc.load_gather(
          vtab_v, [ub + (d + 1)])
      ep1 = td1 - plsc.load_gather(poi_v, [qb + (d + 1)])
      en1 = td1 - plsc.load_gather(poi_v, [rb + (d + 1)])
      accp1 = accp1 + ep1 * ep1
      accn1 = accn1 + en1 * en1
    outp_v[sl] = accp0 + accp1
    outn_v[sl] = accn0 + accn1
    outb_v[sl] = bq - br
    plsc.store_scatter(mask_v, [p], ones16)
    plsc.store_scatter(mask_v, [q], ones16)
    plsc.store_scatter(mask_v, [r], ones16)

  scope.__exit__(None, None, None)
  with jax.named_scope("drain"):
    pass
  pltpu.sync_copy(outp_v, d2p_h.at[pl.ds(base, B_PER_TILE)])
  pltpu.sync_copy(outn_v, d2n_h.at[pl.ds(base, B_PER_TILE)])
  pltpu.sync_copy(outb_v, bd_h.at[pl.ds(base, B_PER_TILE)])
  pltpu.sync_copy(mask_v, mask_h.at[wid])


_sc_kernel = functools.partial(
    pl.kernel,
    out_type=(
        jax.ShapeDtypeStruct((BATCH,), jnp.float32),
        jax.ShapeDtypeStruct((BATCH,), jnp.float32),
        jax.ShapeDtypeStruct((BATCH,), jnp.float32),
        jax.ShapeDtypeStruct((NUM_TILES, N_POI_PAD), jnp.float32),
    ),
    mesh=plsc.VectorSubcoreMesh(core_axis_name="c", subcore_axis_name="s"),
    compiler_params=pltpu.CompilerParams(needs_layout_passes=False),
    scratch_types=[
        pltpu.VMEM((N_POI_PAD * STRIDE,), jnp.float32),
        pltpu.VMEM((N_USERS_PAD * STRIDE,), jnp.float32),
        pltpu.VMEM((N_POI_PAD,), jnp.float32),
        pltpu.VMEM((B_PER_TILE,), jnp.int32),
        pltpu.VMEM((B_PER_TILE,), jnp.int32),
        pltpu.VMEM((B_PER_TILE,), jnp.int32),
        pltpu.VMEM((B_PER_TILE,), jnp.int32),
        pltpu.VMEM((B_PER_TILE,), jnp.float32),
        pltpu.VMEM((B_PER_TILE,), jnp.float32),
        pltpu.VMEM((B_PER_TILE,), jnp.float32),
        pltpu.VMEM((N_POI_PAD,), jnp.float32),
        pltpu.VMEM_SHARED((N_POI_PAD * STRIDE,), jnp.float32),
        pltpu.VMEM_SHARED((N_USERS_PAD * STRIDE,), jnp.float32),
        pltpu.VMEM_SHARED((N_POI_PAD,), jnp.float32),
        pltpu.SemaphoreType.DMA,
    ],
)(_sc_body)


def _tc_body(poi_ref, masks_ref, d2p_ref, d2n_ref, bd_ref, w_ref, obj_ref):
  m = jnp.max(masks_ref[...], axis=0)[:N_POI]
  poi = poi_ref[...]
  n2 = jnp.sum(poi * poi, axis=1)
  denom = jnp.maximum(1.0, jnp.sqrt(n2))
  scale = jnp.where(m > 0.0, 1.0 / denom, 1.0)
  w_ref[...] = poi * scale[:, None]
  obj_ref[...] = bd_ref[...] - jnp.sqrt(d2p_ref[...]) + jnp.sqrt(d2n_ref[...])


def kernel(user_id, prev_id, pos_id, neg_id, poi_weight, user_weight,
           user_global_weight, poi_bias_weight):
  uid = user_id.astype(jnp.int32)
  pid = prev_id.astype(jnp.int32)
  qid = pos_id.astype(jnp.int32)
  nid = neg_id.astype(jnp.int32)
  poi_s = jnp.pad(poi_weight,
                  ((0, N_POI_PAD - N_POI), (0, STRIDE - DIM))).reshape(-1)
  vtab_s = jnp.pad(user_weight + user_global_weight,
                   ((0, N_USERS_PAD - N_USERS), (0, STRIDE - DIM))).reshape(-1)
  bias_p = jnp.pad(poi_bias_weight[:, 0], (0, N_POI_PAD - N_POI))

  d2p, d2n, bd, masks = _sc_kernel(poi_s, vtab_s, bias_p, uid, pid, qid, nid)

  w, obj = pl.pallas_call(
      _tc_body,
      out_shape=(
          jax.ShapeDtypeStruct((N_POI, DIM), jnp.float32),
          jax.ShapeDtypeStruct((BATCH,), jnp.float32),
      ),
  )(poi_weight, masks, d2p, d2n, bd)

  return obj, w
